# Initial kernel scaffold; baseline (speedup 1.0000x reference)
import jax
import jax.numpy as jnp
from jax.experimental import pallas as pl


def kernel(x, edge_index, edge_attr, y, W_emb1, b_emb1, W_c1, b_c1, att_c1, W_c2, b_c2, att_c2, W_c3, b_c3, att_c3, W_emb2, b_emb2, bn_gamma, bn_beta, W_fc1, b_fc1, W_fc4, b_fc4):
    raise NotImplementedError("write your pallas kernel here")


# speedup vs baseline: 30.5502x; 30.5502x over previous
import functools

import jax
import jax.numpy as jnp
from jax import lax
from jax.experimental import pallas as pl
from jax.experimental.pallas import tpu as pltpu
from jax.experimental.pallas import tpu_sc as plsc

_F32 = jnp.float32
_I32 = jnp.int32
_NC = 2
_NS = 16
_L = 16

_SC_PARAMS = pltpu.CompilerParams(
    needs_layout_passes=False, use_tc_tiling_on_sc=False)


def _pre_body(x_ref, kc_ref, k0_ref, ps_ref):
    xb = x_ref[...]
    ps_ref[...] = (jnp.dot(xb, kc_ref[...], preferred_element_type=_F32)
                   + k0_ref[0, :])


def _precompute(x, kc, k0, n, blk):
    return pl.pallas_call(
        _pre_body,
        grid=(n // blk,),
        in_specs=[
            pl.BlockSpec((blk, 128), lambda i: (i, 0)),
            pl.BlockSpec((128, 20), lambda i: (0, 0)),
            pl.BlockSpec((8, 20), lambda i: (0, 0)),
        ],
        out_specs=pl.BlockSpec((blk, 20), lambda i: (i, 0)),
        out_shape=jax.ShapeDtypeStruct((n, 20), _F32),
    )(x, kc, k0)


def _make_pass_a(n2, e):
    W = _NS
    EP = e // W
    CH = 400
    NCH = EP // CH
    SUB = 4
    SB = CH // SUB
    STRIPE = n2 // _NS

    def body(src_hbm, dst_hbm, ea_hbm, ps_hbm, m_hbm, rec_hbm,
             si, di, eav, sr, dr, osv, mv, tsrc, sem):
        sid = lax.axis_index("s")
        w = sid
        srow = sid * STRIPE
        pltpu.sync_copy(ps_hbm.at[pl.ds(srow, STRIPE)],
                        tsrc.at[pl.ds(srow, STRIPE)])
        pltpu.sync_copy(m_hbm, mv)
        zero16 = jnp.zeros((_L,), _F32)

        def _z(i, c):
            osv[i] = zero16
            return c
        lax.fori_loop(0, CH, _z, 0)
        plsc.subcore_barrier()

        ms = [mv[0], mv[1], mv[2]]
        rows0 = lax.iota(_I32, _L)

        def chunk(i, carry):
            base2d = w * (EP // SB) + i * SUB
            pltpu.sync_copy(src_hbm.at[pl.ds(base2d, SUB)], si)
            pltpu.sync_copy(dst_hbm.at[pl.ds(base2d, SUB)], di)
            ebase = w * EP + i * CH
            pltpu.sync_copy(ea_hbm.at[pl.ds(ebase, CH)], eav)
            cps = []
            for j in range(SUB):
                cps.append(pltpu.async_copy(
                    tsrc.at[si.at[j]], sr.at[pl.ds(j * SB, SB)], sem))
                cps.append(pltpu.async_copy(
                    tsrc.at[di.at[j]], dr.at[pl.ds(j * SB, SB)], sem))
            for cp in cps:
                cp.wait()

            def group(g, c2):
                rows = rows0 + g * _L
                for c in range(3):
                    colc = jnp.full((_L,), c, _I32)
                    asv = plsc.load_gather(sr, [rows, colc])
                    adv = plsc.load_gather(dr, [rows, jnp.full((_L,), 3 + c, _I32)])
                    ewv = plsc.load_gather(eav, [rows, colc])
                    v = asv + adv
                    ex = jnp.exp(jnp.maximum(v, 0.2 * v) * ewv - ms[c])
                    plsc.store_scatter(osv, [rows, colc], ex)
                    for j2 in range(2):
                        pv = plsc.load_gather(
                            sr, [rows, jnp.full((_L,), 6 + 2 * c + j2, _I32)])
                        he, ho = plsc.unpack(
                            plsc.bitcast(pv, jnp.bfloat16),
                            format=plsc.PackFormat.INTERLEAVED)
                        cf = 4 + 4 * c + 2 * j2
                        plsc.store_scatter(
                            osv, [rows, jnp.full((_L,), cf, _I32)], he * ex)
                        plsc.store_scatter(
                            osv, [rows, jnp.full((_L,), cf + 1, _I32)], ho * ex)
                return c2
            lax.fori_loop(0, CH // _L, group, 0)
            pltpu.sync_copy(osv, rec_hbm.at[pl.ds(ebase, CH)])
            return carry
        lax.fori_loop(0, NCH, chunk, 0)

    mesh = plsc.VectorSubcoreMesh(
        core_axis_name="c", subcore_axis_name="s", num_cores=1)
    return pl.kernel(
        body, mesh=mesh,
        out_type=jax.ShapeDtypeStruct((e, 16), _F32),
        scratch_types=[
            pltpu.VMEM((SUB, SB), _I32),
            pltpu.VMEM((SUB, SB), _I32),
            pltpu.VMEM((CH, 4), _F32),
            pltpu.VMEM((CH, 16), _F32),
            pltpu.VMEM((CH, 16), _F32),
            pltpu.VMEM((CH, 16), _F32),
            pltpu.VMEM((3, _L), _F32),
            pltpu.VMEM_SHARED((n2, 16), _F32),
            pltpu.SemaphoreType.DMA,
        ],
        compiler_params=_SC_PARAMS,
    )


def _make_pass_b(n2, e):
    W = _NS
    EP = e // W
    CH = 400
    NCH = EP // CH
    SUB = 4
    SB = CH // SUB
    STRIPE = n2 // _NS
    NF = STRIPE // CH
    REM = STRIPE - NF * CH

    def body(dst_hbm, rec_hbm, out_hbm, di, rv, accsh, sem):
        sid = lax.axis_index("s")
        w = sid
        srow = sid * STRIPE
        zero16 = jnp.zeros((_L,), _F32)

        def _z(i, c):
            rv[i] = zero16
            return c
        lax.fori_loop(0, CH, _z, 0)
        for k in range(NF):
            pltpu.sync_copy(rv, accsh.at[pl.ds(srow + k * CH, CH)])
        if REM:
            pltpu.sync_copy(rv.at[pl.ds(0, REM)],
                            accsh.at[pl.ds(srow + NF * CH, REM)])
        plsc.subcore_barrier()

        def chunk(i, carry):
            base2d = w * (EP // SB) + i * SUB
            pltpu.sync_copy(dst_hbm.at[pl.ds(base2d, SUB)], di)
            ebase = w * EP + i * CH
            pltpu.sync_copy(rec_hbm.at[pl.ds(ebase, CH)], rv)
            for j in range(SUB):
                pltpu.sync_copy(rv.at[pl.ds(j * SB, SB)],
                                accsh.at[di.at[j]], add=True)
            return carry
        lax.fori_loop(0, NCH, chunk, 0)
        plsc.subcore_barrier()
        pltpu.sync_copy(accsh.at[pl.ds(srow, STRIPE)],
                        out_hbm.at[pl.ds(srow, STRIPE)])

    mesh = plsc.VectorSubcoreMesh(
        core_axis_name="c", subcore_axis_name="s", num_cores=1)
    return pl.kernel(
        body, mesh=mesh,
        out_type=jax.ShapeDtypeStruct((n2, 16), _F32),
        scratch_types=[
            pltpu.VMEM((SUB, SB), _I32),
            pltpu.VMEM((CH, 16), _F32),
            pltpu.VMEM_SHARED((n2, 16), _F32),
            pltpu.SemaphoreType.DMA,
        ],
        compiler_params=_SC_PARAMS,
    )


def _make_comb(nb, blk):
    def _comb_body(a0_ref, sel_ref, mk_ref, wb_ref, o_ref):
        A = a0_ref[...]
        sel = sel_ref[...]
        mk = mk_ref[...]
        q = jnp.zeros_like(A)
        for c in range(3):
            den = jnp.dot(A, sel[c], preferred_element_type=_F32) + 1e-16
            q = q + (A * mk[c, 0]) / den
        wb = wb_ref[...]
        lane = lax.broadcasted_iota(_I32, (nb, 1, 128), 2)
        bi = lax.broadcasted_iota(_I32, (nb, 1, 128), 0)
        out = jnp.zeros((nb, 1, 128), _F32)
        for b in range(nb):
            qb = q[b * blk:(b + 1) * blk]
            for o in range(4):
                s_bo = jnp.sum(qb * wb[o])
                out = out + jnp.where((bi == b) & (lane == o), s_bo, 0.0)
        o_ref[...] = out

    return pl.pallas_call(
        _comb_body,
        out_shape=jax.ShapeDtypeStruct((nb, 1, 128), _F32),
    )


def kernel(x, edge_index, edge_attr, y, W_emb1, b_emb1, W_c1, b_c1, att_c1,
           W_c2, b_c2, att_c2, W_c3, b_c3, att_c3, W_emb2, b_emb2,
           bn_gamma, bn_beta, W_fc1, b_fc1, W_fc4, b_fc4):
    n = x.shape[0]
    e = edge_index.shape[1]
    bv = y.shape[0]
    npb = n // bv

    T = jnp.zeros((128, 8), _F32)
    T = T.at[7:, 0].set(W_emb1[:, 0])
    T = T.at[jnp.arange(7), 1 + jnp.arange(7)].set(1.0)
    t0 = jnp.zeros((8,), _F32).at[0].set(b_emb1[0])
    W_all = jnp.concatenate([W_c1, W_c2, W_c3], axis=1)
    b_all = jnp.concatenate([b_c1, b_c2, b_c3])
    G = T @ W_all
    g0 = t0 @ W_all + b_all
    atts = [att_c1, att_c2, att_c3]
    kc = jnp.zeros((128, 20), _F32).at[:, 6:18].set(G)
    k0 = jnp.zeros((20,), _F32).at[6:18].set(g0)
    for c in range(3):
        kc = kc.at[:, c].set(G[:, 4 * c:4 * c + 4] @ atts[c][:4])
        k0 = k0.at[c].set(g0[4 * c:4 * c + 4] @ atts[c][:4])
        kc = kc.at[:, 3 + c].set(G[:, 4 * c:4 * c + 4] @ atts[c][4:])
        k0 = k0.at[3 + c].set(g0[4 * c:4 * c + 4] @ atts[c][4:])
    k0full = jnp.tile(k0[None, :], (8, 1))

    p = _precompute(x, kc, k0full, n, 4000)

    M = jnp.maximum(jnp.max(p[:, :3], axis=0)
                    + jnp.max(p[:, 3:6], axis=0), 0.0)
    marr = jnp.repeat(M[:, None], _L, axis=1)

    hu = lax.bitcast_convert_type(p[:, 6:18].astype(jnp.bfloat16),
                                  jnp.uint16).astype(jnp.uint32)
    words = hu[:, 0::2] | (hu[:, 1::2] << 16)
    hp = lax.bitcast_convert_type(words, _F32)
    n2 = _NS * (-(-(n // _NS) // 128) * 128)
    ps2 = jnp.concatenate(
        [p[:, :6], hp, jnp.zeros((n, 4), _F32)], axis=1)
    ps2 = jnp.concatenate([ps2, jnp.zeros((n2 - n, 16), _F32)], axis=0)

    sb = 100
    src2d = edge_index[0].reshape(e // sb, sb)
    dst2d = edge_index[1].reshape(e // sb, sb)

    recs = _make_pass_a(n2, e)(src2d, dst2d, edge_attr, ps2, marr)
    acc = _make_pass_b(n2, e)(dst2d, recs)

    a0 = acc.reshape((n2 * 16) // 128, 128)
    lane = jnp.arange(128)
    k16 = lane // 16
    t16 = lane % 16
    sel = jnp.stack([
        ((lane[:, None] == (k16[None, :] * 16 + c))
         & (t16[None, :] >= 4 + 4 * c) & (t16[None, :] < 8 + 4 * c)
         ).astype(_F32)
        for c in range(3)])
    mk = jnp.stack([
        jnp.tile((((t16 >= 4 + 4 * c) & (t16 < 8 + 4 * c)).astype(_F32)
                  * W_emb2[c, 0])[None, :], (8, 1))
        for c in range(3)])
    wr = W_fc1.reshape(npb, 4, 4)
    rec = jnp.concatenate(
        [jnp.zeros((npb, 4, 4), _F32), wr, wr, wr], axis=1)
    wb = jnp.transpose(rec, (2, 0, 1)).reshape(4, (npb * 16) // 128, 128)

    blk = (npb * 16) // 128
    zres = _make_comb(bv, blk)(a0, sel, mk, wb)

    colsum = jnp.sum(W_fc1, axis=0)
    z = zres[:, 0, :4] + b_emb2[0] * colsum[None, :] + b_fc1[None, :]
    z = jnp.maximum(z, 0.0)
    mu = jnp.mean(z, axis=0)
    var = jnp.var(z, axis=0)
    z = (z - mu) / jnp.sqrt(var + 1e-5) * bn_gamma + bn_beta
    out = z @ W_fc4 + b_fc4
    return (out, jnp.zeros((1,), _F32))


# speedup vs baseline: 33.6115x; 1.1002x over previous
import functools

import jax
import jax.numpy as jnp
from jax import lax
from jax.experimental import pallas as pl
from jax.experimental.pallas import tpu as pltpu
from jax.experimental.pallas import tpu_sc as plsc

_F32 = jnp.float32
_I32 = jnp.int32
_NC = 2
_NS = 16
_L = 16

_SC_PARAMS = pltpu.CompilerParams(
    needs_layout_passes=False, use_tc_tiling_on_sc=False)


def _pre_body(x_ref, kc_ref, k0_ref, ps_ref):
    xb = x_ref[...]
    ps_ref[...] = (jnp.dot(xb, kc_ref[...], preferred_element_type=_F32)
                   + k0_ref[0, :])


def _precompute(x, kc, k0, n, blk):
    return pl.pallas_call(
        _pre_body,
        grid=(n // blk,),
        in_specs=[
            pl.BlockSpec((blk, 128), lambda i: (i, 0)),
            pl.BlockSpec((128, 20), lambda i: (0, 0)),
            pl.BlockSpec((8, 20), lambda i: (0, 0)),
        ],
        out_specs=pl.BlockSpec((blk, 20), lambda i: (i, 0)),
        out_shape=jax.ShapeDtypeStruct((n, 20), _F32),
    )(x, kc, k0)


def _make_pass_a(n2, e):
    W = _NS
    EP = e // W
    CH = 400
    NCH = EP // CH
    SUB = 5
    SB = CH // SUB
    STRIPE = n2 // _NS

    def body(src_hbm, dst_hbm, ea_hbm, ps_hbm, m_hbm, rec_hbm,
             si, di, eav, sr, dr, osv, mv, tsrc, sem):
        sid = lax.axis_index("s")
        w = sid
        srow = sid * STRIPE
        pltpu.sync_copy(ps_hbm.at[pl.ds(srow, STRIPE)],
                        tsrc.at[pl.ds(srow, STRIPE)])
        pltpu.sync_copy(m_hbm, mv)
        zero16 = jnp.zeros((_L,), _F32)

        def _z(i, c):
            osv[i] = zero16
            return c
        lax.fori_loop(0, CH, _z, 0)
        plsc.subcore_barrier()

        ms = [mv[0], mv[1], mv[2]]
        rows0 = lax.iota(_I32, _L)

        def chunk(i, carry):
            ebase = w * EP + i * CH
            pltpu.sync_copy(src_hbm.at[pl.ds(ebase, CH)], si)
            pltpu.sync_copy(dst_hbm.at[pl.ds(ebase, CH)], di)
            pltpu.sync_copy(ea_hbm.at[pl.ds(ebase * 4, CH * 4)], eav)
            cps = []
            for j in range(SUB):
                cps.append(pltpu.async_copy(
                    tsrc.at[si.at[pl.ds(j * SB, SB)]],
                    sr.at[pl.ds(j * SB, SB)], sem))
                cps.append(pltpu.async_copy(
                    tsrc.at[di.at[pl.ds(j * SB, SB)]],
                    dr.at[pl.ds(j * SB, SB)], sem))
            for cp in cps:
                cp.wait()

            def group(g, c2):
                rows = rows0 + g * _L
                for c in range(3):
                    colc = jnp.full((_L,), c, _I32)
                    asv = plsc.load_gather(sr, [rows, colc])
                    adv = plsc.load_gather(dr, [rows, jnp.full((_L,), 3 + c, _I32)])
                    ewv = plsc.load_gather(eav, [rows * 4 + c])
                    v = asv + adv
                    ex = jnp.exp(jnp.maximum(v, 0.2 * v) * ewv - ms[c])
                    plsc.store_scatter(osv, [rows, colc], ex)
                    for j2 in range(2):
                        pv = plsc.load_gather(
                            sr, [rows, jnp.full((_L,), 6 + 2 * c + j2, _I32)])
                        he, ho = plsc.unpack(
                            plsc.bitcast(pv, jnp.bfloat16),
                            format=plsc.PackFormat.INTERLEAVED)
                        cf = 4 + 4 * c + 2 * j2
                        plsc.store_scatter(
                            osv, [rows, jnp.full((_L,), cf, _I32)], he * ex)
                        plsc.store_scatter(
                            osv, [rows, jnp.full((_L,), cf + 1, _I32)], ho * ex)
                return c2
            lax.fori_loop(0, CH // _L, group, 0)
            pltpu.sync_copy(osv, rec_hbm.at[pl.ds(ebase, CH)])
            return carry
        lax.fori_loop(0, NCH, chunk, 0)

    mesh = plsc.VectorSubcoreMesh(
        core_axis_name="c", subcore_axis_name="s", num_cores=1)
    return pl.kernel(
        body, mesh=mesh,
        out_type=jax.ShapeDtypeStruct((e, 16), _F32),
        scratch_types=[
            pltpu.VMEM((CH,), _I32),
            pltpu.VMEM((CH,), _I32),
            pltpu.VMEM((CH * 4,), _F32),
            pltpu.VMEM((CH, 16), _F32),
            pltpu.VMEM((CH, 16), _F32),
            pltpu.VMEM((CH, 16), _F32),
            pltpu.VMEM((3, _L), _F32),
            pltpu.VMEM_SHARED((n2, 16), _F32),
            pltpu.SemaphoreType.DMA,
        ],
        compiler_params=_SC_PARAMS,
    )


def _make_pass_b(n2, e):
    W = _NS
    EP = e // W
    CH = 400
    NCH = EP // CH
    SUB = 4
    SB = CH // SUB
    STRIPE = n2 // _NS
    NF = STRIPE // CH
    REM = STRIPE - NF * CH

    def body(dst_hbm, rec_hbm, out_hbm, di, rv, accsh, sem):
        sid = lax.axis_index("s")
        w = sid
        srow = sid * STRIPE
        zero16 = jnp.zeros((_L,), _F32)

        def _z(i, c):
            rv[i] = zero16
            return c
        lax.fori_loop(0, CH, _z, 0)
        for k in range(NF):
            pltpu.sync_copy(rv, accsh.at[pl.ds(srow + k * CH, CH)])
        if REM:
            pltpu.sync_copy(rv.at[pl.ds(0, REM)],
                            accsh.at[pl.ds(srow + NF * CH, REM)])
        plsc.subcore_barrier()

        def chunk(i, carry):
            base2d = w * (EP // SB) + i * SUB
            pltpu.sync_copy(dst_hbm.at[pl.ds(base2d, SUB)], di)
            ebase = w * EP + i * CH
            pltpu.sync_copy(rec_hbm.at[pl.ds(ebase, CH)], rv)
            for j in range(SUB):
                pltpu.sync_copy(rv.at[pl.ds(j * SB, SB)],
                                accsh.at[di.at[j]], add=True)
            return carry
        lax.fori_loop(0, NCH, chunk, 0)
        plsc.subcore_barrier()
        pltpu.sync_copy(accsh.at[pl.ds(srow, STRIPE)],
                        out_hbm.at[pl.ds(srow, STRIPE)])

    mesh = plsc.VectorSubcoreMesh(
        core_axis_name="c", subcore_axis_name="s", num_cores=1)
    return pl.kernel(
        body, mesh=mesh,
        out_type=jax.ShapeDtypeStruct((n2, 16), _F32),
        scratch_types=[
            pltpu.VMEM((SUB, SB), _I32),
            pltpu.VMEM((CH, 16), _F32),
            pltpu.VMEM_SHARED((n2, 16), _F32),
            pltpu.SemaphoreType.DMA,
        ],
        compiler_params=_SC_PARAMS,
    )


def _make_comb(nb, blk):
    def _comb_body(a0_ref, sel_ref, mk_ref, wb_ref, o_ref):
        A = a0_ref[...]
        sel = sel_ref[...]
        mk = mk_ref[...]
        q = jnp.zeros_like(A)
        for c in range(3):
            den = jnp.dot(A, sel[c], preferred_element_type=_F32) + 1e-16
            q = q + (A * mk[c, 0]) / den
        wb = wb_ref[...]
        lane = lax.broadcasted_iota(_I32, (nb, 1, 128), 2)
        bi = lax.broadcasted_iota(_I32, (nb, 1, 128), 0)
        out = jnp.zeros((nb, 1, 128), _F32)
        for b in range(nb):
            qb = q[b * blk:(b + 1) * blk]
            for o in range(4):
                s_bo = jnp.sum(qb * wb[o])
                out = out + jnp.where((bi == b) & (lane == o), s_bo, 0.0)
        o_ref[...] = out

    return pl.pallas_call(
        _comb_body,
        out_shape=jax.ShapeDtypeStruct((nb, 1, 128), _F32),
    )


def kernel(x, edge_index, edge_attr, y, W_emb1, b_emb1, W_c1, b_c1, att_c1,
           W_c2, b_c2, att_c2, W_c3, b_c3, att_c3, W_emb2, b_emb2,
           bn_gamma, bn_beta, W_fc1, b_fc1, W_fc4, b_fc4):
    n = x.shape[0]
    e = edge_index.shape[1]
    bv = y.shape[0]
    npb = n // bv

    T = jnp.zeros((128, 8), _F32)
    T = T.at[7:, 0].set(W_emb1[:, 0])
    T = T.at[jnp.arange(7), 1 + jnp.arange(7)].set(1.0)
    t0 = jnp.zeros((8,), _F32).at[0].set(b_emb1[0])
    W_all = jnp.concatenate([W_c1, W_c2, W_c3], axis=1)
    b_all = jnp.concatenate([b_c1, b_c2, b_c3])
    G = T @ W_all
    g0 = t0 @ W_all + b_all
    atts = [att_c1, att_c2, att_c3]
    kc = jnp.zeros((128, 20), _F32).at[:, 6:18].set(G)
    k0 = jnp.zeros((20,), _F32).at[6:18].set(g0)
    for c in range(3):
        kc = kc.at[:, c].set(G[:, 4 * c:4 * c + 4] @ atts[c][:4])
        k0 = k0.at[c].set(g0[4 * c:4 * c + 4] @ atts[c][:4])
        kc = kc.at[:, 3 + c].set(G[:, 4 * c:4 * c + 4] @ atts[c][4:])
        k0 = k0.at[3 + c].set(g0[4 * c:4 * c + 4] @ atts[c][4:])
    k0full = jnp.tile(k0[None, :], (8, 1))

    p = _precompute(x, kc, k0full, n, 4000)

    M = jnp.maximum(jnp.max(p[:, :3], axis=0)
                    + jnp.max(p[:, 3:6], axis=0), 0.0)
    marr = jnp.repeat(M[:, None], _L, axis=1)

    hu = lax.bitcast_convert_type(p[:, 6:18].astype(jnp.bfloat16),
                                  jnp.uint16).astype(jnp.uint32)
    words = hu[:, 0::2] | (hu[:, 1::2] << 16)
    hp = lax.bitcast_convert_type(words, _F32)
    n2 = _NS * (-(-(n // _NS) // 128) * 128)
    ps2 = jnp.concatenate(
        [p[:, :6], hp, jnp.zeros((n, 4), _F32)], axis=1)
    ps2 = jnp.concatenate([ps2, jnp.zeros((n2 - n, 16), _F32)], axis=0)

    sb = 100
    srcf = edge_index[0].reshape(e)
    dstf = edge_index[1].reshape(e)
    eaf = edge_attr.reshape(e * 4)
    dst2d = edge_index[1].reshape(e // sb, sb)

    recs = _make_pass_a(n2, e)(srcf, dstf, eaf, ps2, marr)
    acc = _make_pass_b(n2, e)(dst2d, recs)

    a0 = acc.reshape((n2 * 16) // 128, 128)
    lane = jnp.arange(128)
    k16 = lane // 16
    t16 = lane % 16
    sel = jnp.stack([
        ((lane[:, None] == (k16[None, :] * 16 + c))
         & (t16[None, :] >= 4 + 4 * c) & (t16[None, :] < 8 + 4 * c)
         ).astype(_F32)
        for c in range(3)])
    mk = jnp.stack([
        jnp.tile((((t16 >= 4 + 4 * c) & (t16 < 8 + 4 * c)).astype(_F32)
                  * W_emb2[c, 0])[None, :], (8, 1))
        for c in range(3)])
    wr = W_fc1.reshape(npb, 4, 4)
    rec = jnp.concatenate(
        [jnp.zeros((npb, 4, 4), _F32), wr, wr, wr], axis=1)
    wb = jnp.transpose(rec, (2, 0, 1)).reshape(4, (npb * 16) // 128, 128)

    blk = (npb * 16) // 128
    zres = _make_comb(bv, blk)(a0, sel, mk, wb)

    colsum = jnp.sum(W_fc1, axis=0)
    z = zres[:, 0, :4] + b_emb2[0] * colsum[None, :] + b_fc1[None, :]
    z = jnp.maximum(z, 0.0)
    mu = jnp.mean(z, axis=0)
    var = jnp.var(z, axis=0)
    z = (z - mu) / jnp.sqrt(var + 1e-5) * bn_gamma + bn_beta
    out = z @ W_fc4 + b_fc4
    return (out, jnp.zeros((1,), _F32))


# speedup vs baseline: 34.9636x; 1.0402x over previous
import functools

import jax
import jax.numpy as jnp
from jax import lax
from jax.experimental import pallas as pl
from jax.experimental.pallas import tpu as pltpu
from jax.experimental.pallas import tpu_sc as plsc

_F32 = jnp.float32
_I32 = jnp.int32
_NC = 2
_NS = 16
_L = 16

_SC_PARAMS = pltpu.CompilerParams(
    needs_layout_passes=False, use_tc_tiling_on_sc=False)


def _pre_body(x_ref, kc_ref, k0_ref, ps_ref):
    xb = x_ref[...]
    ps_ref[...] = (jnp.dot(xb, kc_ref[...], preferred_element_type=_F32)
                   + k0_ref[0, :])


def _precompute(x, kc, k0, n, blk):
    return pl.pallas_call(
        _pre_body,
        grid=(n // blk,),
        in_specs=[
            pl.BlockSpec((blk, 128), lambda i: (i, 0)),
            pl.BlockSpec((128, 20), lambda i: (0, 0)),
            pl.BlockSpec((8, 20), lambda i: (0, 0)),
        ],
        out_specs=pl.BlockSpec((blk, 20), lambda i: (i, 0)),
        out_shape=jax.ShapeDtypeStruct((n, 20), _F32),
    )(x, kc, k0)


def _eprep_body(ei_ref, ea_ref, s_ref, d_ref, w1_ref, w2_ref, w3_ref):
    eib = ei_ref[...]
    eab = ea_ref[...]
    s_ref[...] = eib[0]
    d_ref[...] = eib[1]
    w1_ref[...] = eab[:, 0]
    w2_ref[...] = eab[:, 1]
    w3_ref[...] = eab[:, 2]


def _edgeprep(ei, ea, e, blk):
    return pl.pallas_call(
        _eprep_body,
        grid=(e // blk,),
        in_specs=[
            pl.BlockSpec((2, blk), lambda i: (0, i)),
            pl.BlockSpec((blk, 4), lambda i: (i, 0)),
        ],
        out_specs=[pl.BlockSpec((blk,), lambda i: (i,))] * 5,
        out_shape=[jax.ShapeDtypeStruct((e,), _I32),
                   jax.ShapeDtypeStruct((e,), _I32),
                   jax.ShapeDtypeStruct((e,), _F32),
                   jax.ShapeDtypeStruct((e,), _F32),
                   jax.ShapeDtypeStruct((e,), _F32)],
    )(ei, ea)


def _make_pass_a(n2, e):
    W = _NS
    EP = e // W
    CH = 400
    NCH = EP // CH
    SUB = 5
    SB = CH // SUB
    STRIPE = n2 // _NS

    def body(src_hbm, dst_hbm, w1_hbm, w2_hbm, w3_hbm, ps_hbm, m_hbm,
             rec_hbm, si, di, ew1, ew2, ew3, sr, dr, osv, mv, tsrc, sem):
        sid = lax.axis_index("s")
        w = sid
        srow = sid * STRIPE
        pltpu.sync_copy(ps_hbm.at[pl.ds(srow, STRIPE)],
                        tsrc.at[pl.ds(srow, STRIPE)])
        pltpu.sync_copy(m_hbm, mv)
        zero16 = jnp.zeros((_L,), _F32)

        def _z(i, c):
            osv[i] = zero16
            return c
        lax.fori_loop(0, CH, _z, 0)
        plsc.subcore_barrier()

        ms = [mv[0], mv[1], mv[2]]
        rows0 = lax.iota(_I32, _L)

        def chunk(i, carry):
            ebase = w * EP + i * CH
            pltpu.sync_copy(src_hbm.at[pl.ds(ebase, CH)], si)
            pltpu.sync_copy(dst_hbm.at[pl.ds(ebase, CH)], di)
            pltpu.sync_copy(w1_hbm.at[pl.ds(ebase, CH)], ew1)
            pltpu.sync_copy(w2_hbm.at[pl.ds(ebase, CH)], ew2)
            pltpu.sync_copy(w3_hbm.at[pl.ds(ebase, CH)], ew3)
            cps = []
            for j in range(SUB):
                cps.append(pltpu.async_copy(
                    tsrc.at[si.at[pl.ds(j * SB, SB)]],
                    sr.at[pl.ds(j * SB, SB)], sem))
                cps.append(pltpu.async_copy(
                    tsrc.at[di.at[pl.ds(j * SB, SB)]],
                    dr.at[pl.ds(j * SB, SB)], sem))
            for cp in cps:
                cp.wait()

            def group(g, c2):
                rows = rows0 + g * _L
                ews = [ew1, ew2, ew3]
                for c in range(3):
                    colc = jnp.full((_L,), c, _I32)
                    asv = plsc.load_gather(sr, [rows, colc])
                    adv = plsc.load_gather(dr, [rows, jnp.full((_L,), 3 + c, _I32)])
                    ewv = ews[c][pl.ds(g * _L, _L)]
                    v = asv + adv
                    ex = jnp.exp(jnp.maximum(v, 0.2 * v) * ewv - ms[c])
                    plsc.store_scatter(osv, [rows, colc], ex)
                    for j2 in range(2):
                        pv = plsc.load_gather(
                            sr, [rows, jnp.full((_L,), 6 + 2 * c + j2, _I32)])
                        he, ho = plsc.unpack(
                            plsc.bitcast(pv, jnp.bfloat16),
                            format=plsc.PackFormat.INTERLEAVED)
                        cf = 4 + 4 * c + 2 * j2
                        plsc.store_scatter(
                            osv, [rows, jnp.full((_L,), cf, _I32)], he * ex)
                        plsc.store_scatter(
                            osv, [rows, jnp.full((_L,), cf + 1, _I32)], ho * ex)
                return c2
            lax.fori_loop(0, CH // _L, group, 0)
            pltpu.sync_copy(osv, rec_hbm.at[pl.ds(ebase, CH)])
            return carry
        lax.fori_loop(0, NCH, chunk, 0)

    mesh = plsc.VectorSubcoreMesh(
        core_axis_name="c", subcore_axis_name="s", num_cores=1)
    return pl.kernel(
        body, mesh=mesh,
        out_type=jax.ShapeDtypeStruct((e, 16), _F32),
        scratch_types=[
            pltpu.VMEM((CH,), _I32),
            pltpu.VMEM((CH,), _I32),
            pltpu.VMEM((CH,), _F32),
            pltpu.VMEM((CH,), _F32),
            pltpu.VMEM((CH,), _F32),
            pltpu.VMEM((CH, 16), _F32),
            pltpu.VMEM((CH, 16), _F32),
            pltpu.VMEM((CH, 16), _F32),
            pltpu.VMEM((3, _L), _F32),
            pltpu.VMEM_SHARED((n2, 16), _F32),
            pltpu.SemaphoreType.DMA,
        ],
        compiler_params=_SC_PARAMS,
    )


def _make_pass_b(n2, e):
    W = _NS
    EP = e // W
    CH = 400
    NCH = EP // CH
    SUB = 5
    SB = CH // SUB
    STRIPE = n2 // _NS
    NF = STRIPE // CH
    REM = STRIPE - NF * CH

    def body(dst_hbm, rec_hbm, out_hbm, di, rv, accsh, sem):
        sid = lax.axis_index("s")
        w = sid
        srow = sid * STRIPE
        zero16 = jnp.zeros((_L,), _F32)

        def _z(i, c):
            rv[i] = zero16
            return c
        lax.fori_loop(0, CH, _z, 0)
        for k in range(NF):
            pltpu.sync_copy(rv, accsh.at[pl.ds(srow + k * CH, CH)])
        if REM:
            pltpu.sync_copy(rv.at[pl.ds(0, REM)],
                            accsh.at[pl.ds(srow + NF * CH, REM)])
        plsc.subcore_barrier()

        def chunk(i, carry):
            ebase = w * EP + i * CH
            pltpu.sync_copy(dst_hbm.at[pl.ds(ebase, CH)], di)
            pltpu.sync_copy(rec_hbm.at[pl.ds(ebase, CH)], rv)
            for j in range(SUB):
                pltpu.sync_copy(rv.at[pl.ds(j * SB, SB)],
                                accsh.at[di.at[pl.ds(j * SB, SB)]], add=True)
            return carry
        lax.fori_loop(0, NCH, chunk, 0)
        plsc.subcore_barrier()
        pltpu.sync_copy(accsh.at[pl.ds(srow, STRIPE)],
                        out_hbm.at[pl.ds(srow, STRIPE)])

    mesh = plsc.VectorSubcoreMesh(
        core_axis_name="c", subcore_axis_name="s", num_cores=1)
    return pl.kernel(
        body, mesh=mesh,
        out_type=jax.ShapeDtypeStruct((n2, 16), _F32),
        scratch_types=[
            pltpu.VMEM((CH,), _I32),
            pltpu.VMEM((CH, 16), _F32),
            pltpu.VMEM_SHARED((n2, 16), _F32),
            pltpu.SemaphoreType.DMA,
        ],
        compiler_params=_SC_PARAMS,
    )


def _make_comb(nb, blk):
    def _comb_body(a0_ref, sel_ref, mk_ref, wb_ref, o_ref):
        A = a0_ref[...]
        sel = sel_ref[...]
        mk = mk_ref[...]
        q = jnp.zeros_like(A)
        for c in range(3):
            den = jnp.dot(A, sel[c], preferred_element_type=_F32) + 1e-16
            q = q + (A * mk[c, 0]) / den
        wb = wb_ref[...]
        lane = lax.broadcasted_iota(_I32, (nb, 1, 128), 2)
        bi = lax.broadcasted_iota(_I32, (nb, 1, 128), 0)
        out = jnp.zeros((nb, 1, 128), _F32)
        for b in range(nb):
            qb = q[b * blk:(b + 1) * blk]
            for o in range(4):
                s_bo = jnp.sum(qb * wb[o])
                out = out + jnp.where((bi == b) & (lane == o), s_bo, 0.0)
        o_ref[...] = out

    return pl.pallas_call(
        _comb_body,
        out_shape=jax.ShapeDtypeStruct((nb, 1, 128), _F32),
    )


def kernel(x, edge_index, edge_attr, y, W_emb1, b_emb1, W_c1, b_c1, att_c1,
           W_c2, b_c2, att_c2, W_c3, b_c3, att_c3, W_emb2, b_emb2,
           bn_gamma, bn_beta, W_fc1, b_fc1, W_fc4, b_fc4):
    n = x.shape[0]
    e = edge_index.shape[1]
    bv = y.shape[0]
    npb = n // bv

    T = jnp.zeros((128, 8), _F32)
    T = T.at[7:, 0].set(W_emb1[:, 0])
    T = T.at[jnp.arange(7), 1 + jnp.arange(7)].set(1.0)
    t0 = jnp.zeros((8,), _F32).at[0].set(b_emb1[0])
    W_all = jnp.concatenate([W_c1, W_c2, W_c3], axis=1)
    b_all = jnp.concatenate([b_c1, b_c2, b_c3])
    G = T @ W_all
    g0 = t0 @ W_all + b_all
    atts = [att_c1, att_c2, att_c3]
    kc = jnp.zeros((128, 20), _F32).at[:, 6:18].set(G)
    k0 = jnp.zeros((20,), _F32).at[6:18].set(g0)
    for c in range(3):
        kc = kc.at[:, c].set(G[:, 4 * c:4 * c + 4] @ atts[c][:4])
        k0 = k0.at[c].set(g0[4 * c:4 * c + 4] @ atts[c][:4])
        kc = kc.at[:, 3 + c].set(G[:, 4 * c:4 * c + 4] @ atts[c][4:])
        k0 = k0.at[3 + c].set(g0[4 * c:4 * c + 4] @ atts[c][4:])
    k0full = jnp.tile(k0[None, :], (8, 1))

    p = _precompute(x, kc, k0full, n, 4000)

    M = jnp.maximum(jnp.max(p[:, :3], axis=0)
                    + jnp.max(p[:, 3:6], axis=0), 0.0)
    marr = jnp.repeat(M[:, None], _L, axis=1)

    hu = lax.bitcast_convert_type(p[:, 6:18].astype(jnp.bfloat16),
                                  jnp.uint16).astype(jnp.uint32)
    words = hu[:, 0::2] | (hu[:, 1::2] << 16)
    hp = lax.bitcast_convert_type(words, _F32)
    n2 = _NS * (-(-(n // _NS) // 128) * 128)
    ps2 = jnp.concatenate(
        [p[:, :6], hp, jnp.zeros((n, 4), _F32)], axis=1)
    ps2 = jnp.concatenate([ps2, jnp.zeros((n2 - n, 16), _F32)], axis=0)

    srcf, dstf, ew1, ew2, ew3 = _edgeprep(edge_index, edge_attr, e, 25600)

    recs = _make_pass_a(n2, e)(srcf, dstf, ew1, ew2, ew3, ps2, marr)
    acc = _make_pass_b(n2, e)(dstf, recs)

    a0 = acc.reshape((n2 * 16) // 128, 128)
    lane = jnp.arange(128)
    k16 = lane // 16
    t16 = lane % 16
    sel = jnp.stack([
        ((lane[:, None] == (k16[None, :] * 16 + c))
         & (t16[None, :] >= 4 + 4 * c) & (t16[None, :] < 8 + 4 * c)
         ).astype(_F32)
        for c in range(3)])
    mk = jnp.stack([
        jnp.tile((((t16 >= 4 + 4 * c) & (t16 < 8 + 4 * c)).astype(_F32)
                  * W_emb2[c, 0])[None, :], (8, 1))
        for c in range(3)])
    wr = W_fc1.reshape(npb, 4, 4)
    rec = jnp.concatenate(
        [jnp.zeros((npb, 4, 4), _F32), wr, wr, wr], axis=1)
    wb = jnp.transpose(rec, (2, 0, 1)).reshape(4, (npb * 16) // 128, 128)

    blk = (npb * 16) // 128
    zres = _make_comb(bv, blk)(a0, sel, mk, wb)

    colsum = jnp.sum(W_fc1, axis=0)
    z = zres[:, 0, :4] + b_emb2[0] * colsum[None, :] + b_fc1[None, :]
    z = jnp.maximum(z, 0.0)
    mu = jnp.mean(z, axis=0)
    var = jnp.var(z, axis=0)
    z = (z - mu) / jnp.sqrt(var + 1e-5) * bn_gamma + bn_beta
    out = z @ W_fc4 + b_fc4
    return (out, jnp.zeros((1,), _F32))
